# baseline (speedup 1.0000x reference)
import functools

import jax
import jax.numpy as jnp
from jax import lax
from jax.experimental import pallas as pl
from jax.experimental.pallas import tpu as pltpu
from jax.experimental.pallas import tpu_sc as plsc

B = 16384
K = 32
N_ROWS = 1000000
NBLK = 7813
NC = 2
NS = 16
NW = NC * NS
BPW = B // NW
L = 16
W16 = 16
NR16 = NBLK * 256
C = 32
RPC = C * K
NCH = BPW // C
NPAIR = NCH // 2
J1 = (NBLK + NW - 1) // NW

_mesh = plsc.VectorSubcoreMesh(core_axis_name="c", subcore_axis_name="s")


@functools.partial(
    pl.kernel,
    mesh=_mesh,
    compiler_params=pltpu.CompilerParams(needs_layout_passes=False),
    out_type=(
        jax.ShapeDtypeStruct((NBLK, K, 128), jnp.float32),
        jax.ShapeDtypeStruct((NBLK, K, 128), jnp.float32),
    ),
    scratch_types=[
        pltpu.VMEM((K, 128), jnp.float32),
        pltpu.VMEM((K, 128), jnp.float32),
        pltpu.VMEM((K, 128), jnp.float32),
        pltpu.VMEM((K, 128), jnp.float32),
        pltpu.SemaphoreType.DMA,
        pltpu.SemaphoreType.DMA,
        pltpu.SemaphoreType.DMA,
        pltpu.SemaphoreType.DMA,
    ],
)
def _relayout(wu_t_hbm, wi_t_hbm, vu_hbm, vi_hbm,
              ua, ub, ia, ib, ra, rb, wa, wb):
    wid = lax.axis_index("s") * NC + lax.axis_index("c")

    def rd(i, bu, bi, sem):
        off = pl.multiple_of(i * 128, 128)
        pltpu.async_copy(wu_t_hbm.at[:, pl.ds(off, 128)], bu, sem)
        pltpu.async_copy(wi_t_hbm.at[:, pl.ds(off, 128)], bi, sem)

    def rd_drain(bu, bi, sem):
        pltpu.make_async_copy(wu_t_hbm.at[:, pl.ds(0, 128)], bu, sem).wait()
        pltpu.make_async_copy(wi_t_hbm.at[:, pl.ds(0, 128)], bi, sem).wait()

    def wr(i, bu, bi, sem):
        pltpu.async_copy(bu, vu_hbm.at[i], sem)
        pltpu.async_copy(bi, vi_hbm.at[i], sem)

    def wr_drain(bu, bi, sem):
        pltpu.make_async_copy(bu, vu_hbm.at[0], sem).wait()
        pltpu.make_async_copy(bi, vi_hbm.at[0], sem).wait()

    def step(j, carry):
        i0 = wid + (2 * j) * NW
        i1 = wid + (2 * j + 1) * NW

        @pl.when(i0 < NBLK)
        def _():
            rd(i0, ua, ia, ra)

        @pl.when(i1 < NBLK)
        def _():
            rd(i1, ub, ib, rb)

        @pl.when(i0 < NBLK)
        def _():
            rd_drain(ua, ia, ra)
            wr(i0, ua, ia, wa)

        @pl.when(i1 < NBLK)
        def _():
            rd_drain(ub, ib, rb)
            wr(i1, ub, ib, wb)

        @pl.when(i0 < NBLK)
        def _():
            wr_drain(ua, ia, wa)

        @pl.when(i1 < NBLK)
        def _():
            wr_drain(ub, ib, wb)

        return carry

    lax.fori_loop(0, (J1 + 1) // 2, step, 0)


@functools.partial(
    pl.kernel,
    mesh=_mesh,
    compiler_params=pltpu.CompilerParams(
        needs_layout_passes=False, use_tc_tiling_on_sc=False
    ),
    out_type=jax.ShapeDtypeStruct((B,), jnp.float32),
    scratch_types=[
        pltpu.VMEM((BPW,), jnp.int32),
        pltpu.VMEM((BPW,), jnp.int32),
        pltpu.VMEM((BPW * K,), jnp.int32),
        pltpu.VMEM((BPW * K,), jnp.int32),
        pltpu.VMEM((RPC, W16), jnp.float32),
        pltpu.VMEM((RPC, W16), jnp.float32),
        pltpu.VMEM((RPC, W16), jnp.float32),
        pltpu.VMEM((RPC, W16), jnp.float32),
        pltpu.VMEM((BPW,), jnp.float32),
        pltpu.SemaphoreType.DMA,
        pltpu.SemaphoreType.DMA,
    ],
)
def _pmf_sc(user_hbm, item_hbm, vu_hbm, vi_hbm, out_hbm,
            uidx, iidx, rlu, rli, gu0, gu1, gi0, gi1, oacc, sem0, sem1):
    wid = lax.axis_index("s") * NC + lax.axis_index("c")
    base = wid * BPW

    pltpu.sync_copy(user_hbm.at[pl.ds(base, BPW)], uidx)
    pltpu.sync_copy(item_hbm.at[pl.ds(base, BPW)], iidx)

    def build(g, carry):
        u = uidx[pl.ds(g * L, L)]
        v = iidx[pl.ds(g * L, L)]
        ubase = (lax.shift_right_logical(u, 7) * 256
                 + jnp.bitwise_and(lax.shift_right_logical(u, 4), 7))
        vbase = (lax.shift_right_logical(v, 7) * 256
                 + jnp.bitwise_and(lax.shift_right_logical(v, 4), 7))
        pos = (g // 2) * RPC + (g % 2) * L
        for k in range(K):
            rlu[pl.ds(pos + k * C, L)] = ubase + (k * 8)
            rli[pl.ds(pos + k * C, L)] = vbase + (k * 8)
        return carry

    lax.fori_loop(0, BPW // L, build, 0)

    def fire(c, bu, bi, sem):
        sl = pl.ds(c * RPC, RPC)
        pltpu.async_copy(vu_hbm.at[rlu.at[sl]], bu, sem)
        pltpu.async_copy(vi_hbm.at[rli.at[sl]], bi, sem)

    def drain(bu, bi, sem):
        pltpu.make_async_copy(vu_hbm.at[rlu.at[pl.ds(0, RPC)]], bu, sem).wait()
        pltpu.make_async_copy(vi_hbm.at[rli.at[pl.ds(0, RPC)]], bi, sem).wait()

    def compute(c, bu, bi):
        for g2 in range(C // L):
            isl = pl.ds(c * C + g2 * L, L)
            ulane = jnp.bitwise_and(uidx[isl], W16 - 1)
            ilane = jnp.bitwise_and(iidx[isl], W16 - 1)
            acc = jnp.zeros((L,), jnp.float32)
            for k in range(K):
                rows = k * C + g2 * L + lax.iota(jnp.int32, L)
                uval = plsc.load_gather(bu, [rows, ulane])
                ival = plsc.load_gather(bi, [rows, ilane])
                acc = acc + uval * ival
            oacc[isl] = acc

    fire(0, gu0, gi0, sem0)

    def pair(p, carry):
        c0 = p * 2
        fire(c0 + 1, gu1, gi1, sem1)
        drain(gu0, gi0, sem0)
        compute(c0, gu0, gi0)

        @pl.when(p < NPAIR - 1)
        def _():
            fire(c0 + 2, gu0, gi0, sem0)

        drain(gu1, gi1, sem1)
        compute(c0 + 1, gu1, gi1)
        return carry

    lax.fori_loop(0, NPAIR, pair, 0)

    pltpu.sync_copy(oacc, out_hbm.at[pl.ds(base, BPW)])


def kernel(user, item, W_user, W_item):
    vu, vi = _relayout(W_user.T, W_item.T)
    vu16 = vu.reshape(NR16, W16)
    vi16 = vi.reshape(NR16, W16)
    return _pmf_sc(user, item, vu16, vi16)


# speedup vs baseline: 1.3117x; 1.3117x over previous
import functools

import jax
import jax.numpy as jnp
from jax import lax
from jax.experimental import pallas as pl
from jax.experimental.pallas import tpu as pltpu
from jax.experimental.pallas import tpu_sc as plsc

B = 16384
K = 32
KH = K // 2
N_ROWS = 1000000
NC = 2
NS = 16
NW = NC * NS
BPW = B // NW
C = 16
NCH = BPW // C
L = 16


_mesh = plsc.VectorSubcoreMesh(core_axis_name="c", subcore_axis_name="s")


@functools.partial(
    pl.kernel,
    mesh=_mesh,
    compiler_params=pltpu.CompilerParams(needs_layout_passes=False),
    out_type=jax.ShapeDtypeStruct((B,), jnp.float32),
    scratch_types=[
        pltpu.VMEM((BPW,), jnp.int32),
        pltpu.VMEM((BPW,), jnp.int32),
        pltpu.VMEM((C, KH, 128), jnp.float32),
        pltpu.VMEM((C, KH, 128), jnp.float32),
        pltpu.VMEM((BPW,), jnp.float32),
        pltpu.SemaphoreType.DMA,
    ],
)
def _pmf_sc(user_hbm, item_hbm, wu_t_hbm, wi_t_hbm, out_hbm,
            uvec, ivec, ublk, iblk, oacc, sem):
    wid = lax.axis_index("s") * NC + lax.axis_index("c")
    base = wid * BPW

    pltpu.sync_copy(user_hbm.at[pl.ds(base, BPW)], uvec)
    pltpu.sync_copy(item_hbm.at[pl.ds(base, BPW)], ivec)

    def chunk(c, carry):
        ulane = jnp.bitwise_and(uvec[pl.ds(c * C, L)], 127)
        ilane = jnp.bitwise_and(ivec[pl.ds(c * C, L)], 127)
        ubase = lax.shift_left(
            lax.shift_right_logical(uvec[pl.ds(c * C, L)], 7), 7)
        ibase = lax.shift_left(
            lax.shift_right_logical(ivec[pl.ds(c * C, L)], 7), 7)
        sel = lax.iota(jnp.int32, L)
        zero = jnp.zeros((L,), jnp.int32)
        acc = jnp.zeros((L,), jnp.float32)

        for kh in range(K // KH):
            copies = []
            for i in range(C):
                ub = pl.multiple_of(
                    jnp.sum(jnp.where(sel == i, ubase, zero)), 128)
                ib = pl.multiple_of(
                    jnp.sum(jnp.where(sel == i, ibase, zero)), 128)
                copies.append(pltpu.async_copy(
                    wu_t_hbm.at[pl.ds(kh * KH, KH), pl.ds(ub, 128)],
                    ublk.at[i], sem))
                copies.append(pltpu.async_copy(
                    wi_t_hbm.at[pl.ds(kh * KH, KH), pl.ds(ib, 128)],
                    iblk.at[i], sem))
            for cp in copies:
                cp.wait()

            for kk in range(KH):
                kvec = jnp.full((L,), kk, jnp.int32)
                u = plsc.load_gather(ublk, [sel, kvec, ulane])
                v = plsc.load_gather(iblk, [sel, kvec, ilane])
                acc = acc + u * v

        oacc[pl.ds(c * C, L)] = acc
        return carry

    lax.fori_loop(0, NCH, chunk, 0)

    pltpu.sync_copy(oacc, out_hbm.at[pl.ds(base, BPW)])


def kernel(user, item, W_user, W_item):
    return _pmf_sc(user, item, W_user.T, W_item.T)
